# Initial kernel scaffold; baseline (speedup 1.0000x reference)
import jax
import jax.numpy as jnp
from jax.experimental import pallas as pl


def kernel(x, pe):
    raise NotImplementedError("write your pallas kernel here")


# speedup vs baseline: 1.9401x; 1.9401x over previous
import jax
import jax.numpy as jnp
from jax.experimental import pallas as pl


def _add_body(pe_ref, x_ref, o_ref):
    o_ref[...] = x_ref[...] + pe_ref[...]


def kernel(x, pe):
    b, t, d = x.shape
    bt = 512
    grid = (t // bt, b)
    return pl.pallas_call(
        _add_body,
        grid=grid,
        in_specs=[
            pl.BlockSpec((bt, d), lambda j, i: (j, 0)),
            pl.BlockSpec((1, bt, d), lambda j, i: (i, j, 0)),
        ],
        out_specs=pl.BlockSpec((1, bt, d), lambda j, i: (i, j, 0)),
        out_shape=jax.ShapeDtypeStruct(x.shape, x.dtype),
    )(pe[:t], x)
